# baseline (speedup 1.0000x reference)
import functools

import jax
import jax.numpy as jnp
from jax.experimental import pallas as pl

N = 256
E = 8192
D = 1280
H = 128


def _graph_kernel(ei_ref, ef_ref, mi_ref, feat0_ref,
                  Wg_ref, bg_ref, Wg1_ref, bg1_ref,
                  Wm1_ref, bm1_ref, Wm2_ref, bm2_ref,
                  W1_ref, b1h_ref, b1v_ref,
                  xbf_ref, s_ref, sT_ref, mask_out_ref):
    f32 = jnp.float32
    src = ei_ref[0:1, :]
    dst = ei_ref[1:2, :]
    ew = 1.0 / (ef_ref[...] ** 2 + 1e-6)

    node_iota = jax.lax.broadcasted_iota(jnp.int32, (N, E), 0)
    oh_src_t = (src == node_iota).astype(f32)
    oh_dst_w = (dst == node_iota).astype(f32) * ew

    A = jax.lax.dot_general(oh_dst_w, oh_src_t,
                            (((1,), (1,)), ((), ())),
                            preferred_element_type=f32)

    feat0 = feat0_ref[...]

    def dense(x, w_ref, b_ref):
        return jax.lax.dot_general(x, w_ref[...],
                                   (((1,), (1,)), ((), ())),
                                   preferred_element_type=f32) + b_ref[...]

    h = dense(feat0 + jnp.dot(A, feat0, preferred_element_type=f32),
              Wg_ref, bg_ref)
    X = dense(h + jnp.dot(A, h, preferred_element_type=f32),
              Wg1_ref, bg1_ref) + feat0
    xbf_ref[...] = X.astype(jnp.bfloat16)

    X2 = X * X
    s_ref[...] = jax.lax.dot_general(X2, W1_ref[...],
                                     (((1,), (1,)), ((), ())),
                                     preferred_element_type=f32) + b1h_ref[...]
    sT_ref[...] = jax.lax.dot_general(W1_ref[...], X2,
                                      (((1,), (1,)), ((), ())),
                                      preferred_element_type=f32) + b1v_ref[...]

    mi = mi_ref[...]
    mask_iota = jax.lax.broadcasted_iota(jnp.int32, (N, 32), 0)
    oh_mask_t = (mi == mask_iota).astype(f32)
    m = jax.lax.dot_general(oh_mask_t, X,
                            (((0,), (0,)), ((), ())),
                            preferred_element_type=f32)
    m = jax.nn.relu(dense(m, Wm1_ref, bm1_ref))
    mask_out_ref[...] = jnp.tanh(dense(m, Wm2_ref, bm2_ref))


def _dis_kernel(xj_ref, x_all_ref, W1m2_ref, sj_ref, sT_ref,
                W2p_ref, b2p_ref, out_ref, *, bj):
    f32 = jnp.float32
    xj = xj_ref[...]
    Z = (xj[:, None, :] * W1m2_ref[...][None, :, :]).reshape(bj * H, D)
    BT = jax.lax.dot_general(Z, x_all_ref[...],
                             (((1,), (1,)), ((), ())),
                             preferred_element_type=f32)
    B3 = BT.reshape(bj, H, N)
    y1 = sj_ref[...][:, :, None] + sT_ref[...][None, :, :] + B3
    y1 = jax.nn.relu(y1).astype(jnp.bfloat16)
    out = jax.lax.dot_general(W2p_ref[...], y1,
                              (((1,), (1,)), ((), ())),
                              preferred_element_type=f32)
    out = out + b2p_ref[...][:, :, None]
    for jj in range(bj):
        out_ref[:, pl.ds(jj * N, N)] = out[:, jj, :]


def kernel(lm_embedding, node_feat, edge_feat, edge_index, mask_index,
           W_gin, b_gin, W_gin1, b_gin1,
           W_dis1, b_dis1, W_dis2, b_dis2,
           W_mask1, b_mask1, W_mask2, b_mask2):
    feat0 = jnp.concatenate([lm_embedding[0, 1:-1, :], node_feat], axis=1)

    xbf, s, sT, mask_pred = pl.pallas_call(
        _graph_kernel,
        out_shape=[
            jax.ShapeDtypeStruct((N, D), jnp.bfloat16),
            jax.ShapeDtypeStruct((N, H), jnp.float32),
            jax.ShapeDtypeStruct((H, N), jnp.float32),
            jax.ShapeDtypeStruct((32, 2), jnp.float32),
        ],
    )(edge_index.astype(jnp.int32),
      edge_feat.reshape(1, E),
      mask_index.reshape(1, 32).astype(jnp.int32),
      feat0,
      W_gin, b_gin.reshape(1, D), W_gin1, b_gin1.reshape(1, D),
      W_mask1, b_mask1.reshape(1, H), W_mask2, b_mask2.reshape(1, 2),
      W_dis1, (0.5 * b_dis1).reshape(1, H), (0.5 * b_dis1).reshape(H, 1))

    W1m2 = (-2.0 * W_dis1).astype(jnp.bfloat16)
    W2p = jnp.concatenate([W_dis2, jnp.zeros((2, H), jnp.float32)],
                          axis=0).astype(jnp.bfloat16)
    b2p = jnp.concatenate([b_dis2, jnp.zeros((2,), jnp.float32)]).reshape(32, 1)

    BJ = 32
    outT = pl.pallas_call(
        functools.partial(_dis_kernel, bj=BJ),
        grid=(N // BJ,),
        in_specs=[
            pl.BlockSpec((BJ, D), lambda j: (j, 0)),
            pl.BlockSpec((N, D), lambda j: (0, 0)),
            pl.BlockSpec((H, D), lambda j: (0, 0)),
            pl.BlockSpec((BJ, H), lambda j: (j, 0)),
            pl.BlockSpec((H, N), lambda j: (0, 0)),
            pl.BlockSpec((32, H), lambda j: (0, 0)),
            pl.BlockSpec((32, 1), lambda j: (0, 0)),
        ],
        out_specs=pl.BlockSpec((32, BJ * N), lambda j: (0, j)),
        out_shape=jax.ShapeDtypeStruct((32, N * N), jnp.float32),
    )(xbf, xbf, W1m2, s, sT, W2p, b2p)

    dis_pred = outT[:30].T
    return (dis_pred, mask_pred)


# speedup vs baseline: 1.0313x; 1.0313x over previous
import functools

import jax
import jax.numpy as jnp
from jax.experimental import pallas as pl
from jax.experimental.pallas import tpu as pltpu

N = 256
E = 8192
D = 1280
H = 128


def _fused_kernel(ei_ref, ef_ref, mi_ref, feat0_ref,
                  Wg_ref, bg_ref, Wg1_ref, bg1_ref,
                  Wm1_ref, bm1_ref, Wm2_ref, bm2_ref,
                  W1_ref, b1h_ref, b1v_ref,
                  W1m2_ref, W2p_ref, b2p_ref,
                  out_ref, mask_out_ref,
                  xbf_s, s_s, sT_s, *, bj):
    f32 = jnp.float32
    j = pl.program_id(0)

    @pl.when(j == 0)
    def _graph_stage():
        src = ei_ref[0:1, :]
        dst = ei_ref[1:2, :]
        ew = 1.0 / (ef_ref[...] ** 2 + 1e-6)

        node_iota = jax.lax.broadcasted_iota(jnp.int32, (N, E), 0)
        oh_src_t = (src == node_iota).astype(f32)
        oh_dst_w = (dst == node_iota).astype(f32) * ew

        A = jax.lax.dot_general(oh_dst_w, oh_src_t,
                                (((1,), (1,)), ((), ())),
                                preferred_element_type=f32)

        feat0 = feat0_ref[...]

        def dense(x, w_ref, b_ref):
            return jax.lax.dot_general(x, w_ref[...],
                                       (((1,), (1,)), ((), ())),
                                       preferred_element_type=f32) + b_ref[...]

        h = dense(feat0 + jnp.dot(A, feat0, preferred_element_type=f32),
                  Wg_ref, bg_ref)
        X = dense(h + jnp.dot(A, h, preferred_element_type=f32),
                  Wg1_ref, bg1_ref) + feat0
        xbf_s[...] = X.astype(jnp.bfloat16)

        X2 = X * X
        s_s[...] = jax.lax.dot_general(X2, W1_ref[...],
                                       (((1,), (1,)), ((), ())),
                                       preferred_element_type=f32) + b1h_ref[...]
        sT_s[...] = jax.lax.dot_general(W1_ref[...], X2,
                                        (((1,), (1,)), ((), ())),
                                        preferred_element_type=f32) + b1v_ref[...]

        mi = mi_ref[...]
        mask_iota = jax.lax.broadcasted_iota(jnp.int32, (N, 32), 0)
        oh_mask_t = (mi == mask_iota).astype(f32)
        m = jax.lax.dot_general(oh_mask_t, X,
                                (((0,), (0,)), ((), ())),
                                preferred_element_type=f32)
        m = jax.nn.relu(dense(m, Wm1_ref, bm1_ref))
        mask_out_ref[...] = jnp.tanh(dense(m, Wm2_ref, bm2_ref))

    xj = xbf_s[pl.ds(j * bj, bj), :]
    Z = (xj[:, None, :] * W1m2_ref[...][None, :, :]).reshape(bj * H, D)
    BT = jax.lax.dot_general(Z, xbf_s[...],
                             (((1,), (1,)), ((), ())),
                             preferred_element_type=f32)
    B3 = BT.reshape(bj, H, N)
    sj = s_s[pl.ds(j * bj, bj), :]
    y1 = sj[:, :, None] + sT_s[...][None, :, :] + B3
    y1 = jax.nn.relu(y1).astype(jnp.bfloat16)
    out = jax.lax.dot_general(W2p_ref[...], y1,
                              (((1,), (1,)), ((), ())),
                              preferred_element_type=f32)
    out = out + b2p_ref[...][:, :, None]
    for jj in range(bj):
        out_ref[:, pl.ds(jj * N, N)] = out[:, jj, :]


def kernel(lm_embedding, node_feat, edge_feat, edge_index, mask_index,
           W_gin, b_gin, W_gin1, b_gin1,
           W_dis1, b_dis1, W_dis2, b_dis2,
           W_mask1, b_mask1, W_mask2, b_mask2):
    feat0 = jnp.concatenate([lm_embedding[0, 1:-1, :], node_feat], axis=1)

    W1m2 = (-2.0 * W_dis1).astype(jnp.bfloat16)
    W2p = jnp.concatenate([W_dis2, jnp.zeros((2, H), jnp.float32)],
                          axis=0).astype(jnp.bfloat16)
    b2p = jnp.concatenate([b_dis2, jnp.zeros((2,), jnp.float32)]).reshape(32, 1)

    BJ = 32
    c = lambda j: (0, 0)
    outT, mask_pred = pl.pallas_call(
        functools.partial(_fused_kernel, bj=BJ),
        grid=(N // BJ,),
        in_specs=[
            pl.BlockSpec((2, E), c),
            pl.BlockSpec((1, E), c),
            pl.BlockSpec((1, 32), c),
            pl.BlockSpec((N, D), c),
            pl.BlockSpec((D, D), c),
            pl.BlockSpec((1, D), c),
            pl.BlockSpec((D, D), c),
            pl.BlockSpec((1, D), c),
            pl.BlockSpec((H, D), c),
            pl.BlockSpec((1, H), c),
            pl.BlockSpec((2, H), c),
            pl.BlockSpec((1, 2), c),
            pl.BlockSpec((H, D), c),
            pl.BlockSpec((1, H), c),
            pl.BlockSpec((H, 1), c),
            pl.BlockSpec((H, D), c),
            pl.BlockSpec((32, H), c),
            pl.BlockSpec((32, 1), c),
        ],
        out_specs=[
            pl.BlockSpec((32, BJ * N), lambda j: (0, j)),
            pl.BlockSpec((32, 2), c),
        ],
        out_shape=[
            jax.ShapeDtypeStruct((32, N * N), jnp.float32),
            jax.ShapeDtypeStruct((32, 2), jnp.float32),
        ],
        scratch_shapes=[
            pltpu.VMEM((N, D), jnp.bfloat16),
            pltpu.VMEM((N, H), jnp.float32),
            pltpu.VMEM((H, N), jnp.float32),
        ],
    )(edge_index.astype(jnp.int32),
      edge_feat.reshape(1, E),
      mask_index.reshape(1, 32).astype(jnp.int32),
      feat0,
      W_gin, b_gin.reshape(1, D), W_gin1, b_gin1.reshape(1, D),
      W_mask1, b_mask1.reshape(1, H), W_mask2, b_mask2.reshape(1, 2),
      W_dis1, (0.5 * b_dis1).reshape(1, H), (0.5 * b_dis1).reshape(H, 1),
      W1m2, W2p, b2p)

    dis_pred = outT[:30].T
    return (dis_pred, mask_pred)


# speedup vs baseline: 1.1685x; 1.1330x over previous
import functools

import jax
import jax.numpy as jnp
from jax.experimental import pallas as pl
from jax.experimental.pallas import tpu as pltpu

N = 256
E = 8192
D = 1280
D_LM = 1024
H = 128


def _fused_kernel(ei_ref, ef_ref, mi_ref, lm_ref, nf_ref,
                  Wg_ref, bg_ref, Wg1_ref, bg1_ref,
                  Wm1_ref, bm1_ref, Wm2_ref, bm2_ref,
                  W1_ref, b1_ref, W2_ref,
                  out_ref, mask_out_ref,
                  xbf_s, s_s, sT_s, W1m2_s, *, bj):
    f32 = jnp.float32
    j = pl.program_id(0)

    @pl.when(j == 0)
    def _graph_stage():
        src = ei_ref[0:1, :]
        dst = ei_ref[1:2, :]
        ew = 1.0 / (ef_ref[...] ** 2 + 1e-6)

        node_iota = jax.lax.broadcasted_iota(jnp.int32, (N, E), 0)
        oh_src_t = (src == node_iota).astype(f32)
        oh_dst_w = (dst == node_iota).astype(f32) * ew

        A = jax.lax.dot_general(oh_dst_w, oh_src_t,
                                (((1,), (1,)), ((), ())),
                                preferred_element_type=f32)

        feat0 = jnp.concatenate([lm_ref[1:N + 1, :], nf_ref[...]], axis=1)

        def dense(x, w_ref, b_ref):
            return jax.lax.dot_general(x, w_ref[...],
                                       (((1,), (1,)), ((), ())),
                                       preferred_element_type=f32) + b_ref[...]

        h = dense(feat0 + jnp.dot(A, feat0, preferred_element_type=f32),
                  Wg_ref, bg_ref)
        X = dense(h + jnp.dot(A, h, preferred_element_type=f32),
                  Wg1_ref, bg1_ref) + feat0
        xbf_s[...] = X.astype(jnp.bfloat16)
        W1m2_s[...] = (-2.0 * W1_ref[...]).astype(jnp.bfloat16)

        X2 = X * X
        s_s[...] = jax.lax.dot_general(X2, W1_ref[...],
                                       (((1,), (1,)), ((), ())),
                                       preferred_element_type=f32) + b1_ref[...]
        sT_s[...] = jax.lax.dot_general(W1_ref[...], X2,
                                        (((1,), (1,)), ((), ())),
                                        preferred_element_type=f32)

        mi = mi_ref[...]
        mask_iota = jax.lax.broadcasted_iota(jnp.int32, (N, 32), 0)
        oh_mask_t = (mi == mask_iota).astype(f32)
        m = jax.lax.dot_general(oh_mask_t, X,
                                (((0,), (0,)), ((), ())),
                                preferred_element_type=f32)
        m = jax.nn.relu(dense(m, Wm1_ref, bm1_ref))
        mask_out_ref[...] = jnp.tanh(dense(m, Wm2_ref, bm2_ref))

    xj = xbf_s[pl.ds(j * bj, bj), :]
    Z = (xj[:, None, :] * W1m2_s[...][None, :, :]).reshape(bj * H, D)
    BT = jax.lax.dot_general(Z, xbf_s[...],
                             (((1,), (1,)), ((), ())),
                             preferred_element_type=f32)
    B3 = BT.reshape(bj, H, N)
    sj = s_s[pl.ds(j * bj, bj), :]
    y1 = sj[:, :, None] + sT_s[...][None, :, :] + B3
    y1 = jax.nn.relu(y1).astype(jnp.bfloat16)
    out = jax.lax.dot_general(W2_ref[...].astype(jnp.bfloat16), y1,
                              (((1,), (1,)), ((), ())),
                              preferred_element_type=f32)
    for jj in range(bj):
        out_ref[:, pl.ds(jj * N, N)] = out[:, jj, :]


def kernel(lm_embedding, node_feat, edge_feat, edge_index, mask_index,
           W_gin, b_gin, W_gin1, b_gin1,
           W_dis1, b_dis1, W_dis2, b_dis2,
           W_mask1, b_mask1, W_mask2, b_mask2):
    BJ = 32
    c = lambda j: (0, 0)
    outT, mask_pred = pl.pallas_call(
        functools.partial(_fused_kernel, bj=BJ),
        grid=(N // BJ,),
        in_specs=[
            pl.BlockSpec((2, E), c),
            pl.BlockSpec((1, E), c),
            pl.BlockSpec((1, 32), c),
            pl.BlockSpec((N + 2, D_LM), c),
            pl.BlockSpec((N, D - D_LM), c),
            pl.BlockSpec((D, D), c),
            pl.BlockSpec((1, D), c),
            pl.BlockSpec((D, D), c),
            pl.BlockSpec((1, D), c),
            pl.BlockSpec((H, D), c),
            pl.BlockSpec((1, H), c),
            pl.BlockSpec((2, H), c),
            pl.BlockSpec((1, 2), c),
            pl.BlockSpec((H, D), c),
            pl.BlockSpec((1, H), c),
            pl.BlockSpec((30, H), c),
        ],
        out_specs=[
            pl.BlockSpec((30, BJ * N), lambda j: (0, j)),
            pl.BlockSpec((32, 2), c),
        ],
        out_shape=[
            jax.ShapeDtypeStruct((30, N * N), jnp.float32),
            jax.ShapeDtypeStruct((32, 2), jnp.float32),
        ],
        scratch_shapes=[
            pltpu.VMEM((N, D), jnp.bfloat16),
            pltpu.VMEM((N, H), jnp.float32),
            pltpu.VMEM((H, N), jnp.float32),
            pltpu.VMEM((H, D), jnp.bfloat16),
        ],
    )(edge_index, edge_feat.reshape(1, E),
      mask_index.reshape(1, 32),
      lm_embedding.reshape(N + 2, D_LM), node_feat,
      W_gin, b_gin.reshape(1, D), W_gin1, b_gin1.reshape(1, D),
      W_mask1, b_mask1.reshape(1, H), W_mask2, b_mask2.reshape(1, 2),
      W_dis1, b_dis1.reshape(1, H), W_dis2)

    dis_pred = outT.T + b_dis2[None, :]
    return (dis_pred, mask_pred)
